# baseline (speedup 1.0000x reference)
import functools

import jax
import jax.numpy as jnp
from jax import lax
from jax.experimental import pallas as pl
from jax.experimental.pallas import tpu as pltpu
from jax.experimental.pallas import tpu_sc as plsc

EMB_DIM = 32
BATCH = 16384
NC, NS = 2, 16
NW = NC * NS
BPW = BATCH // NW
CHUNK = 128
NCHUNK = BPW // CHUNK

_MESH = plsc.VectorSubcoreMesh(
    core_axis_name="c", subcore_axis_name="s", num_cores=NC, num_subcores=NS)


def _sc_gather_body(ug_hbm, ig_hbm, um_hbm, im_hbm, uid_hbm, iid_hbm,
                    out_ug, out_ig, out_um, out_im,
                    uidx_v, iidx_v, r_ug, r_ig, r_um, r_im, sem):
    wid = lax.axis_index("s") * NC + lax.axis_index("c")
    row0 = wid * NCHUNK
    pltpu.sync_copy(uid_hbm.at[pl.ds(row0, NCHUNK)], uidx_v)
    pltpu.sync_copy(iid_hbm.at[pl.ds(row0, NCHUNK)], iidx_v)
    copies = []
    for j in range(NCHUNK):
        copies.append(pltpu.async_copy(ug_hbm.at[uidx_v.at[j]], r_ug.at[j], sem))
        copies.append(pltpu.async_copy(ig_hbm.at[iidx_v.at[j]], r_ig.at[j], sem))
        copies.append(pltpu.async_copy(um_hbm.at[uidx_v.at[j]], r_um.at[j], sem))
        copies.append(pltpu.async_copy(im_hbm.at[iidx_v.at[j]], r_im.at[j], sem))
    for c in copies:
        c.wait()
    pltpu.sync_copy(r_ug, out_ug.at[pl.ds(row0, NCHUNK)])
    pltpu.sync_copy(r_ig, out_ig.at[pl.ds(row0, NCHUNK)])
    pltpu.sync_copy(r_um, out_um.at[pl.ds(row0, NCHUNK)])
    pltpu.sync_copy(r_im, out_im.at[pl.ds(row0, NCHUNK)])


_ROWS3D = (BATCH // CHUNK, CHUNK, EMB_DIM)

_sc_gather = pl.kernel(
    _sc_gather_body,
    out_type=[jax.ShapeDtypeStruct(_ROWS3D, jnp.float32)] * 4,
    mesh=_MESH,
    scratch_types=[
        pltpu.VMEM((NCHUNK, CHUNK), jnp.int32),
        pltpu.VMEM((NCHUNK, CHUNK), jnp.int32),
        pltpu.VMEM((NCHUNK, CHUNK, EMB_DIM), jnp.float32),
        pltpu.VMEM((NCHUNK, CHUNK, EMB_DIM), jnp.float32),
        pltpu.VMEM((NCHUNK, CHUNK, EMB_DIM), jnp.float32),
        pltpu.VMEM((NCHUNK, CHUNK, EMB_DIM), jnp.float32),
        pltpu.SemaphoreType.DMA,
    ],
    compiler_params=pltpu.CompilerParams(use_tc_tiling_on_sc=False),
)


def _mlp_body(ug, ig, um, im, w1u, w1i, b1, w2t, b2, w3t, b3, wfg, wfh, bf,
              out):
    f32 = jnp.float32
    h = jnp.dot(um[...], w1u[...], preferred_element_type=f32)
    h += jnp.dot(im[...], w1i[...], preferred_element_type=f32)
    h = jnp.maximum(h + b1[...], 0.0)
    h = jnp.maximum(jnp.dot(h, w2t[...], preferred_element_type=f32) + b2[...], 0.0)
    h = jnp.maximum(jnp.dot(h, w3t[...], preferred_element_type=f32) + b3[...], 0.0)
    gmf = ug[...] * ig[...]
    logit = (jnp.dot(gmf, wfg[...], preferred_element_type=f32)
             + jnp.dot(h, wfh[...], preferred_element_type=f32) + bf[...])
    out[...] = jax.nn.sigmoid(logit)


_BS = 2048


def _mlp_call(ug, ig, um, im, w1u, w1i, b1, w2t, b2, w3t, b3, wfg, wfh, bf):
    row_spec = pl.BlockSpec((_BS, EMB_DIM), lambda i: (i, 0))
    full = pl.BlockSpec(index_map=lambda i: (0, 0))
    return pl.pallas_call(
        _mlp_body,
        grid=(BATCH // _BS,),
        in_specs=[row_spec, row_spec, row_spec, row_spec] + [full] * 10,
        out_specs=pl.BlockSpec((_BS, 1), lambda i: (i, 0)),
        out_shape=jax.ShapeDtypeStruct((BATCH, 1), jnp.float32),
    )(ug, ig, um, im, w1u, w1i, b1, w2t, b2, w3t, b3, wfg, wfh, bf)


def kernel(user_emb_gmf, item_emb_gmf, user_emb_mlp, item_emb_mlp,
           W1, b1, W2, b2, W3, b3, Wf, bf, user_ids, item_ids):
    uid = user_ids.astype(jnp.int32).reshape(BATCH // CHUNK, CHUNK)
    iid = item_ids.astype(jnp.int32).reshape(BATCH // CHUNK, CHUNK)
    ug, ig, um, im = _sc_gather(user_emb_gmf, item_emb_gmf,
                                user_emb_mlp, item_emb_mlp, uid, iid)
    ug = ug.reshape(BATCH, EMB_DIM)
    ig = ig.reshape(BATCH, EMB_DIM)
    um = um.reshape(BATCH, EMB_DIM)
    im = im.reshape(BATCH, EMB_DIM)
    w1u = W1[:, :EMB_DIM].T
    w1i = W1[:, EMB_DIM:].T
    wfg = Wf[:, :EMB_DIM].T
    wfh = Wf[:, EMB_DIM:].T
    return _mlp_call(ug, ig, um, im, w1u, w1i, b1.reshape(1, -1),
                     W2.T, b2.reshape(1, -1), W3.T, b3.reshape(1, -1),
                     wfg, wfh, bf.reshape(1, 1))
